# baseline (speedup 1.0000x reference)
import jax
import jax.numpy as jnp
from jax.experimental import pallas as pl

_B, _N, _M, _D, _K = 1024, 26, 26, 128, 10
_BB = 8

_HIGH = jax.lax.Precision.HIGHEST


def _dot(a, b):
    return jax.lax.dot(a, b, precision=_HIGH, preferred_element_type=jnp.float32)


def _attn_kernel(f_ref, c_ref, wq_ref, wk_ref, out_ref):
    wqT = wq_ref[...].T
    wkT = wk_ref[...].T

    m_iota = jax.lax.broadcasted_iota(jnp.int32, (_N, _M), 1)
    r_rows = jax.lax.broadcasted_iota(jnp.int32, (_N * _K, _N), 0)
    r_cols = jax.lax.broadcasted_iota(jnp.int32, (_N * _K, _N), 1)
    R = (r_cols == r_rows // _K).astype(jnp.float32)
    E = [(r_rows == r_cols * _K + k).astype(jnp.float32) for k in range(_K)]

    for b in range(_BB):
        fb = f_ref[b]
        cb = c_ref[b]
        qb = jax.lax.dot(fb, wqT, preferred_element_type=jnp.float32)
        kb = jax.lax.dot(cb, wkT, preferred_element_type=jnp.float32)
        w = jax.lax.dot_general(
            qb, kb, (((1,), (1,)), ((), ())),
            preferred_element_type=jnp.float32)

        vals, idxs = [], []
        wcur = w
        for _ in range(_K):
            mx = jnp.max(wcur, axis=1, keepdims=True)
            am = jnp.min(jnp.where(wcur == mx, m_iota, _M), axis=1,
                         keepdims=True)
            vals.append(mx)
            idxs.append(am)
            wcur = jnp.where(m_iota == am, -jnp.inf, wcur)

        exps = [jnp.exp(v - vals[0]) for v in vals]
        inv = 1.0 / sum(exps)

        P = None
        for k in range(_K):
            Pk = jnp.where(m_iota == idxs[k], exps[k] * inv, 0.0)
            term = _dot(E[k], Pk)
            P = term if P is None else P + term

        G = _dot(P, cb)
        F = _dot(R, fb)
        out_ref[b] = G * F


def kernel(featureVec, contextVec, Wq, Wk):
    return pl.pallas_call(
        _attn_kernel,
        grid=(_B // _BB,),
        in_specs=[
            pl.BlockSpec((_BB, _N, _D), lambda i: (i, 0, 0)),
            pl.BlockSpec((_BB, _M, _D), lambda i: (i, 0, 0)),
            pl.BlockSpec((_D, _D), lambda i: (0, 0)),
            pl.BlockSpec((_D, _D), lambda i: (0, 0)),
        ],
        out_specs=pl.BlockSpec((_BB, _N * _K, _D), lambda i: (i, 0, 0)),
        out_shape=jax.ShapeDtypeStruct((_B, _N * _K, _D), jnp.float32),
    )(featureVec, contextVec, Wq, Wk)


# speedup vs baseline: 2.2148x; 2.2148x over previous
import jax
import jax.numpy as jnp
from jax.experimental import pallas as pl

_B, _N, _M, _D, _K = 1024, 26, 26, 128, 10
_BB = 8

def _dot(a, b):
    return jax.lax.dot(a, b, preferred_element_type=jnp.float32)


def _attn_kernel(f_ref, c_ref, wq_ref, wk_ref, out_ref):
    wqT = wq_ref[...].T
    wkT = wk_ref[...].T

    m_iota = jax.lax.broadcasted_iota(jnp.int32, (_N, _M), 1)
    r_rows = jax.lax.broadcasted_iota(jnp.int32, (_N * _K, _N), 0)
    r_cols = jax.lax.broadcasted_iota(jnp.int32, (_N * _K, _N), 1)
    R = (r_cols == r_rows // _K).astype(jnp.float32)
    E = [(r_rows == r_cols * _K + k).astype(jnp.float32) for k in range(_K)]

    for b in range(_BB):
        fb = f_ref[b]
        cb = c_ref[b]
        qb = jax.lax.dot(fb, wqT, preferred_element_type=jnp.float32)
        kb = jax.lax.dot(cb, wkT, preferred_element_type=jnp.float32)
        w = jax.lax.dot_general(
            qb, kb, (((1,), (1,)), ((), ())),
            preferred_element_type=jnp.float32)

        vals, idxs = [], []
        wcur = w
        for _ in range(_K):
            mx = jnp.max(wcur, axis=1, keepdims=True)
            am = jnp.min(jnp.where(wcur == mx, m_iota, _M), axis=1,
                         keepdims=True)
            vals.append(mx)
            idxs.append(am)
            wcur = jnp.where(m_iota == am, -jnp.inf, wcur)

        exps = [jnp.exp(v - vals[0]) for v in vals]
        inv = 1.0 / sum(exps)

        P = None
        for k in range(_K):
            Pk = jnp.where(m_iota == idxs[k], exps[k] * inv, 0.0)
            term = _dot(E[k], Pk)
            P = term if P is None else P + term

        G = _dot(P, cb)
        F = _dot(R, fb)
        out_ref[b] = G * F


def kernel(featureVec, contextVec, Wq, Wk):
    return pl.pallas_call(
        _attn_kernel,
        grid=(_B // _BB,),
        in_specs=[
            pl.BlockSpec((_BB, _N, _D), lambda i: (i, 0, 0)),
            pl.BlockSpec((_BB, _M, _D), lambda i: (i, 0, 0)),
            pl.BlockSpec((_D, _D), lambda i: (0, 0)),
            pl.BlockSpec((_D, _D), lambda i: (0, 0)),
        ],
        out_specs=pl.BlockSpec((_BB, _N * _K, _D), lambda i: (i, 0, 0)),
        out_shape=jax.ShapeDtypeStruct((_B, _N * _K, _D), jnp.float32),
    )(featureVec, contextVec, Wq, Wk)


# speedup vs baseline: 9.5049x; 4.2916x over previous
import jax
import jax.numpy as jnp
from jax.experimental import pallas as pl

_B, _N, _M, _D, _K = 1024, 26, 26, 128, 10
_BB = 8
_L = _BB * _N


def _dot(a, b):
    return jax.lax.dot(a, b, preferred_element_type=jnp.float32)


def _attn_kernel(f_ref, c_ref, wq_ref, wk_ref, out_ref):
    wqT = wq_ref[...].T
    wkT = wk_ref[...].T
    f = f_ref[...]
    c = c_ref[...]

    q = _dot(f, wqT)
    k = _dot(c, wkT)
    W = jax.lax.dot_general(
        q, k, (((1,), (1,)), ((), ())),
        preferred_element_type=jnp.float32)

    S = jnp.concatenate(
        [W[_N * b:_N * (b + 1), _M * b:_M * (b + 1)] for b in range(_BB)],
        axis=0)

    m_iota = jax.lax.broadcasted_iota(jnp.int32, (_L, _M), 1)
    vals, idxs = [], []
    wcur = S
    for _ in range(_K):
        mx = jnp.max(wcur, axis=1, keepdims=True)
        am = jnp.min(jnp.where(wcur == mx, m_iota, _M), axis=1,
                     keepdims=True)
        vals.append(mx)
        idxs.append(am)
        wcur = jnp.where(m_iota == am, -jnp.inf, wcur)

    exps = [jnp.exp(v - vals[0]) for v in vals]
    inv = 1.0 / sum(exps)

    row_iota = jax.lax.broadcasted_iota(jnp.int32, (_L, 1), 0)
    boff = (row_iota // _N) * _M
    l_iota = jax.lax.broadcasted_iota(jnp.int32, (_L, _L), 1)
    outs = []
    for kk in range(_K):
        Pk = jnp.where(l_iota == idxs[kk] + boff, exps[kk] * inv, 0.0)
        Gk = _dot(Pk, c)
        outs.append(Gk * f)
    out_ref[...] = jnp.stack(outs, axis=1)


def kernel(featureVec, contextVec, Wq, Wk):
    f2 = featureVec.reshape(_B * _N, _D)
    c2 = contextVec.reshape(_B * _M, _D)
    out = pl.pallas_call(
        _attn_kernel,
        grid=(_B // _BB,),
        in_specs=[
            pl.BlockSpec((_L, _D), lambda i: (i, 0)),
            pl.BlockSpec((_L, _D), lambda i: (i, 0)),
            pl.BlockSpec((_D, _D), lambda i: (0, 0)),
            pl.BlockSpec((_D, _D), lambda i: (0, 0)),
        ],
        out_specs=pl.BlockSpec((_L, _K, _D), lambda i: (i, 0, 0)),
        out_shape=jax.ShapeDtypeStruct((_B * _N, _K, _D), jnp.float32),
    )(f2, c2, Wq, Wk)
    return out.reshape(_B, _N * _K, _D)


# speedup vs baseline: 9.9854x; 1.0506x over previous
import jax
import jax.numpy as jnp
from jax.experimental import pallas as pl

_B, _N, _M, _D, _K = 1024, 26, 26, 128, 10
_BB = 8
_L = _BB * _N


def _dot(a, b):
    return jax.lax.dot(a, b, preferred_element_type=jnp.float32)


def _attn_kernel(f_ref, c_ref, wq_ref, wk_ref, out_ref):
    wqT = wq_ref[...].T
    wkT = wk_ref[...].T
    f = f_ref[...]
    c = c_ref[...]

    q = _dot(f, wqT)
    k = _dot(c, wkT)
    W = jax.lax.dot_general(
        q, k, (((1,), (1,)), ((), ())),
        preferred_element_type=jnp.float32)

    S = jnp.concatenate(
        [W[_N * b:_N * (b + 1), _M * b:_M * (b + 1)] for b in range(_BB)],
        axis=0)

    m_iota = jax.lax.broadcasted_iota(jnp.int32, (_L, _M), 1)
    vals, idxs = [], []
    wcur = S
    for _ in range(_K):
        mx = jnp.max(wcur, axis=1, keepdims=True)
        am = jnp.min(jnp.where(wcur == mx, m_iota, _M), axis=1,
                     keepdims=True)
        vals.append(mx)
        idxs.append(am)
        wcur = jnp.where(m_iota == am, -jnp.inf, wcur)

    exps = [jnp.exp(v - vals[0]) for v in vals]
    inv = 1.0 / sum(exps)

    row_iota = jax.lax.broadcasted_iota(jnp.int32, (_L, 1), 0)
    boff = (row_iota // _N) * _M
    l_iota = jax.lax.broadcasted_iota(jnp.int32, (_L, _L), 1)
    for kk in range(_K):
        Pk = jnp.where(l_iota == idxs[kk] + boff, exps[kk] * inv, 0.0)
        Gk = _dot(Pk, c)
        out_ref[:, kk, :] = Gk * f


def kernel(featureVec, contextVec, Wq, Wk):
    f2 = featureVec.reshape(_B * _N, _D)
    c2 = contextVec.reshape(_B * _M, _D)
    out = pl.pallas_call(
        _attn_kernel,
        grid=(_B // _BB,),
        in_specs=[
            pl.BlockSpec((_L, _D), lambda i: (i, 0)),
            pl.BlockSpec((_L, _D), lambda i: (i, 0)),
            pl.BlockSpec((_D, _D), lambda i: (0, 0)),
            pl.BlockSpec((_D, _D), lambda i: (0, 0)),
        ],
        out_specs=pl.BlockSpec((_L, _K, _D), lambda i: (i, 0, 0)),
        out_shape=jax.ShapeDtypeStruct((_B * _N, _K, _D), jnp.float32),
    )(f2, c2, Wq, Wk)
    return out.reshape(_B, _N * _K, _D)


# speedup vs baseline: 12.9797x; 1.2999x over previous
import jax
import jax.numpy as jnp
from jax.experimental import pallas as pl

_B, _N, _M, _D, _K = 1024, 26, 26, 128, 10
_BB = 8
_L = _BB * _N


def _dot(a, b):
    return jax.lax.dot(a, b, preferred_element_type=jnp.float32)


def _attn_kernel(f_ref, c_ref, wq_ref, wk_ref, out_ref):
    wqT = wq_ref[...].T
    wkT = wk_ref[...].T
    f = f_ref[...]
    c = c_ref[...]

    q = _dot(f, wqT)
    k = _dot(c, wkT)
    W = jax.lax.dot_general(
        q, k, (((1,), (1,)), ((), ())),
        preferred_element_type=jnp.float32)

    S = jnp.concatenate(
        [W[_N * b:_N * (b + 1), _M * b:_M * (b + 1)] for b in range(_BB)],
        axis=0)

    vals = []
    wcur = S
    for _ in range(_K):
        mx = jnp.max(wcur, axis=1, keepdims=True)
        vals.append(mx)
        wcur = jnp.where(wcur == mx, -jnp.inf, wcur)

    exps = [jnp.exp(v - vals[0]) for v in vals]
    inv = 1.0 / sum(exps)

    row_iota = jax.lax.broadcasted_iota(jnp.int32, (_L, _L), 0)
    l_iota = jax.lax.broadcasted_iota(jnp.int32, (_L, _L), 1)
    Wm = jnp.where(row_iota // _N == l_iota // _M, W, -jnp.inf)
    for kk in range(_K):
        Pk = jnp.where(Wm == vals[kk], exps[kk] * inv, 0.0)
        Gk = _dot(Pk, c)
        out_ref[:, kk, :] = Gk * f


def kernel(featureVec, contextVec, Wq, Wk):
    f2 = featureVec.reshape(_B * _N, _D)
    c2 = contextVec.reshape(_B * _M, _D)
    out = pl.pallas_call(
        _attn_kernel,
        grid=(_B // _BB,),
        in_specs=[
            pl.BlockSpec((_L, _D), lambda i: (i, 0)),
            pl.BlockSpec((_L, _D), lambda i: (i, 0)),
            pl.BlockSpec((_D, _D), lambda i: (0, 0)),
            pl.BlockSpec((_D, _D), lambda i: (0, 0)),
        ],
        out_specs=pl.BlockSpec((_L, _K, _D), lambda i: (i, 0, 0)),
        out_shape=jax.ShapeDtypeStruct((_B * _N, _K, _D), jnp.float32),
    )(f2, c2, Wq, Wk)
    return out.reshape(_B, _N * _K, _D)


# speedup vs baseline: 22.1015x; 1.7028x over previous
import jax
import jax.numpy as jnp
from jax.experimental import pallas as pl

_B, _N, _M, _D, _K = 1024, 26, 26, 128, 10
_BB = 8
_L = _BB * _N


def _dot(a, b):
    return jax.lax.dot(a, b, preferred_element_type=jnp.float32)


def _attn_kernel(f_ref, c_ref, wq_ref, wk_ref, out_ref):
    wqT = wq_ref[...].T
    wkT = wk_ref[...].T
    f = jnp.concatenate([f_ref[b] for b in range(_BB)], axis=0)
    c = jnp.concatenate([c_ref[b] for b in range(_BB)], axis=0)

    q = _dot(f, wqT)
    k = _dot(c, wkT)
    W = jax.lax.dot_general(
        q, k, (((1,), (1,)), ((), ())),
        preferred_element_type=jnp.float32)

    S = jnp.concatenate(
        [W[_N * b:_N * (b + 1), _M * b:_M * (b + 1)] for b in range(_BB)],
        axis=0)

    vals = []
    wcur = S
    for _ in range(_K):
        mx = jnp.max(wcur, axis=1, keepdims=True)
        vals.append(mx)
        wcur = jnp.where(wcur == mx, -jnp.inf, wcur)

    exps = [jnp.exp(v - vals[0]) for v in vals]
    inv = 1.0 / sum(exps)

    row_iota = jax.lax.broadcasted_iota(jnp.int32, (_L, _L), 0)
    l_iota = jax.lax.broadcasted_iota(jnp.int32, (_L, _L), 1)
    Wm = jnp.where(row_iota // _N == l_iota // _M, W, -jnp.inf)
    for kk in range(_K):
        Pk = jnp.where(Wm == vals[kk], exps[kk] * inv, 0.0)
        Gk = _dot(Pk, c)
        Ok = Gk * f
        for b in range(_BB):
            out_ref[pl.ds(b, 1), pl.Slice(kk, _N, _K), :] = (
                Ok[_N * b:_N * (b + 1), :].reshape(1, _N, _D))


def kernel(featureVec, contextVec, Wq, Wk):
    return pl.pallas_call(
        _attn_kernel,
        grid=(_B // _BB,),
        in_specs=[
            pl.BlockSpec((_BB, _N, _D), lambda i: (i, 0, 0)),
            pl.BlockSpec((_BB, _M, _D), lambda i: (i, 0, 0)),
            pl.BlockSpec((_D, _D), lambda i: (0, 0)),
            pl.BlockSpec((_D, _D), lambda i: (0, 0)),
        ],
        out_specs=pl.BlockSpec((_BB, _N * _K, _D), lambda i: (i, 0, 0)),
        out_shape=jax.ShapeDtypeStruct((_B, _N * _K, _D), jnp.float32),
    )(featureVec, contextVec, Wq, Wk)
